# Initial kernel scaffold; baseline (speedup 1.0000x reference)
import jax
import jax.numpy as jnp
from jax.experimental import pallas as pl


def kernel(indices, tables, w0, b0, w1, b1):
    raise NotImplementedError("write your pallas kernel here")


# speedup vs baseline: 1.0616x; 1.0616x over previous
import functools

import jax
import jax.numpy as jnp
from jax import lax
from jax.experimental import pallas as pl
from jax.experimental.pallas import tpu as pltpu
from jax.experimental.pallas import tpu_sc as plsc

F = 26
V = 100000
E = 16
B = 16384
NP = F * (F - 1) // 2
NIN = F * E
BT = B * F

NC = 2
NS = 16
NW = NC * NS
B_PER_W = BT // NW
CHUNK = 128
N_CHUNKS = B_PER_W // CHUNK
INFLIGHT = 8
N_ROUNDS = N_CHUNKS // INFLIGHT
ROWS_BUF = INFLIGHT * CHUNK


def _sc_gather(table_h, idx_h, out_h, idx_v, rows_v, sem):
    wid = lax.axis_index("s") * NC + lax.axis_index("c")
    pltpu.sync_copy(idx_h.at[pl.ds(wid * N_CHUNKS, N_CHUNKS)], idx_v)
    base = wid * B_PER_W

    def round_body(g, carry):
        handles = []
        for j in range(INFLIGHT):
            handles.append(
                pltpu.async_copy(
                    table_h.at[idx_v.at[g * INFLIGHT + j]],
                    rows_v.at[pl.ds(j * CHUNK, CHUNK)],
                    sem,
                )
            )
        for h in handles:
            h.wait()
        pltpu.sync_copy(rows_v, out_h.at[pl.ds(base + g * ROWS_BUF, ROWS_BUF)])
        return carry

    lax.fori_loop(0, N_ROUNDS, round_body, 0)


def _gather_call(table_flat, idx_flat):
    mesh = plsc.VectorSubcoreMesh(core_axis_name="c", subcore_axis_name="s")
    return pl.kernel(
        _sc_gather,
        mesh=mesh,
        out_type=jax.ShapeDtypeStruct((BT, E), jnp.float32),
        scratch_types=[
            pltpu.VMEM((N_CHUNKS, CHUNK), jnp.int32),
            pltpu.VMEM((ROWS_BUF, E), jnp.float32),
            pltpu.SemaphoreType.DMA,
        ],
        compiler_params=pltpu.CompilerParams(use_tc_tiling_on_sc=False),
    )(table_flat, idx_flat)


BB = 512


def _tc_body(x_ref, w0at_ref, w0bt_ref, b0_ref, w1_ref, b1_ref, o_ref, ipt_ref):
    x = x_ref[:]
    xt = x.T
    off = 0
    for i in range(F - 1):
        n = F - 1 - i
        a = xt[i * E:(i + 1) * E, :]
        rest = xt[(i + 1) * E:, :]
        prod = rest.reshape(n, E, BB) * a[None, :, :]
        ipt_ref[pl.ds(off, n), :] = jnp.sum(prod, axis=1)
        off += n
    ht = (
        jnp.dot(w0at_ref[:], xt, preferred_element_type=jnp.float32)
        + jnp.dot(w0bt_ref[:], ipt_ref[:], preferred_element_type=jnp.float32)
        + b0_ref[:]
    )
    ht = jnp.maximum(ht, 0.0)
    logit = jnp.dot(w1_ref[:], ht, preferred_element_type=jnp.float32) + b1_ref[:]
    o_ref[:] = (1.0 / (1.0 + jnp.exp(-logit))).reshape(1, 1, BB)


def _mlp_call(x, w0at, w0bt, b0c, w1r, b1c):
    grid = (B // BB,)
    return pl.pallas_call(
        _tc_body,
        grid=grid,
        in_specs=[
            pl.BlockSpec((BB, NIN), lambda i: (i, 0)),
            pl.BlockSpec((400, NIN), lambda i: (0, 0)),
            pl.BlockSpec((400, NP), lambda i: (0, 0)),
            pl.BlockSpec((400, 1), lambda i: (0, 0)),
            pl.BlockSpec((1, 400), lambda i: (0, 0)),
            pl.BlockSpec((1, 1), lambda i: (0, 0)),
        ],
        out_specs=pl.BlockSpec((1, 1, BB), lambda i: (i, 0, 0)),
        out_shape=jax.ShapeDtypeStruct((B // BB, 1, BB), jnp.float32),
        scratch_shapes=[pltpu.VMEM((NP, BB), jnp.float32)],
    )(x, w0at, w0bt, b0c, w1r, b1c)


def kernel(indices, tables, w0, b0, w1, b1):
    table_flat = tables.reshape(F * V, E)
    idx_flat = (indices + (jnp.arange(F, dtype=jnp.int32) * V)[None, :]).reshape(
        NW * N_CHUNKS, CHUNK
    )
    rows = _gather_call(table_flat, idx_flat)
    x = rows.reshape(B, NIN)
    w0at = w0[:NIN].T
    w0bt = w0[NIN:].T
    y2d = _mlp_call(x, w0at, w0bt, b0.reshape(400, 1), w1.reshape(1, 400),
                    b1.reshape(1, 1))
    return y2d.reshape(B)


# speedup vs baseline: 4.1448x; 3.9044x over previous
import functools

import jax
import jax.numpy as jnp
from jax import lax
from jax.experimental import pallas as pl
from jax.experimental.pallas import tpu as pltpu
from jax.experimental.pallas import tpu_sc as plsc

F = 26
V = 100000
E = 16
B = 16384
NP = F * (F - 1) // 2
NIN = F * E

NC = 2
NS = 16
NW = NC * NS
ROWS_PER_W = NIN // NW
OUT_CHUNK = 4096


def _sc_gather(table_h, idxt_h, out_h, plane_v, idx_v, out_v, sem, osem):
    wid = lax.axis_index("s") * NC + lax.axis_index("c")

    def unit(u, carry):
        r = wid * ROWS_PER_W + u
        f = r // E
        plane_cp = pltpu.async_copy(table_h.at[r, :], plane_v, sem)
        idx_cp = pltpu.async_copy(idxt_h.at[f, :], idx_v, sem)
        plane_cp.wait()
        idx_cp.wait()

        def chunk(c, carry2):
            def vec(k, carry3):
                base = c * OUT_CHUNK + k * E
                iv = idx_v[pl.ds(base, E)]
                out_v[pl.ds(k * E, E)] = plsc.load_gather(plane_v, [iv])
                return carry3

            lax.fori_loop(0, OUT_CHUNK // E, vec, 0, unroll=8)
            pltpu.async_copy(
                out_v, out_h.at[r, pl.ds(c * OUT_CHUNK, OUT_CHUNK)], osem
            ).wait()
            return carry2

        lax.fori_loop(0, B // OUT_CHUNK, chunk, 0)
        return carry

    lax.fori_loop(0, ROWS_PER_W, unit, 0)


def _gather_call(table_t, idx_t):
    mesh = plsc.VectorSubcoreMesh(core_axis_name="c", subcore_axis_name="s")
    return pl.kernel(
        _sc_gather,
        mesh=mesh,
        out_type=jax.ShapeDtypeStruct((NIN, B), jnp.float32),
        scratch_types=[
            pltpu.VMEM((V,), jnp.float32),
            pltpu.VMEM((B,), jnp.int32),
            pltpu.VMEM((OUT_CHUNK,), jnp.float32),
            pltpu.SemaphoreType.DMA,
            pltpu.SemaphoreType.DMA,
        ],
        compiler_params=pltpu.CompilerParams(
            use_tc_tiling_on_sc=True, needs_layout_passes=False
        ),
    )(table_t, idx_t)


BB = 512


def _tc_body(x_ref, w0at_ref, w0bt_ref, b0_ref, w1_ref, b1_ref, o_ref, ipt_ref):
    xt = x_ref[:]
    off = 0
    for i in range(F - 1):
        n = F - 1 - i
        a = xt[i * E:(i + 1) * E, :]
        rest = xt[(i + 1) * E:, :]
        prod = rest.reshape(n, E, BB) * a[None, :, :]
        ipt_ref[pl.ds(off, n), :] = jnp.sum(prod, axis=1)
        off += n
    ht = (
        jnp.dot(w0at_ref[:], xt, preferred_element_type=jnp.float32)
        + jnp.dot(w0bt_ref[:], ipt_ref[:], preferred_element_type=jnp.float32)
        + b0_ref[:]
    )
    ht = jnp.maximum(ht, 0.0)
    logit = jnp.dot(w1_ref[:], ht, preferred_element_type=jnp.float32) + b1_ref[:]
    o_ref[:] = (1.0 / (1.0 + jnp.exp(-logit))).reshape(1, 1, BB)


def _mlp_call(xt, w0at, w0bt, b0c, w1r, b1c):
    grid = (B // BB,)
    return pl.pallas_call(
        _tc_body,
        grid=grid,
        in_specs=[
            pl.BlockSpec((NIN, BB), lambda i: (0, i)),
            pl.BlockSpec((400, NIN), lambda i: (0, 0)),
            pl.BlockSpec((400, NP), lambda i: (0, 0)),
            pl.BlockSpec((400, 1), lambda i: (0, 0)),
            pl.BlockSpec((1, 400), lambda i: (0, 0)),
            pl.BlockSpec((1, 1), lambda i: (0, 0)),
        ],
        out_specs=pl.BlockSpec((1, 1, BB), lambda i: (i, 0, 0)),
        out_shape=jax.ShapeDtypeStruct((B // BB, 1, BB), jnp.float32),
        scratch_shapes=[pltpu.VMEM((NP, BB), jnp.float32)],
    )(xt, w0at, w0bt, b0c, w1r, b1c)


def kernel(indices, tables, w0, b0, w1, b1):
    table_t = jnp.transpose(tables, (0, 2, 1)).reshape(NIN, V)
    idx_t = indices.T
    xt = _gather_call(table_t, idx_t)
    w0at = w0[:NIN].T
    w0bt = w0[NIN:].T
    y2d = _mlp_call(xt, w0at, w0bt, b0.reshape(400, 1), w1.reshape(1, 400),
                    b1.reshape(1, 1))
    return y2d.reshape(B)
